# baseline (speedup 1.0000x reference)
import jax
import jax.numpy as jnp
from jax.experimental import pallas as pl
from jax.experimental.pallas import tpu as pltpu

BT = 1024
E = 16
D = 4096


def _router_kernel(x_ref, w_ref, weights_ref, topw_ref, tope_ref):
    xb = x_ref[...]
    w = w_ref[...]
    logits = jax.lax.dot_general(
        xb, w, (((1,), (1,)), ((), ())),
        preferred_element_type=jnp.float32)

    m1 = jnp.max(logits, axis=-1, keepdims=True)
    s = jnp.exp(logits - m1)
    denom = jnp.sum(s, axis=-1, keepdims=True)
    weights = s / denom
    weights_ref[...] = weights

    iota = jax.lax.broadcasted_iota(jnp.int32, weights.shape, 1)
    w1 = jnp.max(weights, axis=-1, keepdims=True)
    a1 = jnp.min(jnp.where(weights == w1, iota, E), axis=-1, keepdims=True)
    masked = jnp.where(iota == a1, -jnp.inf, weights)
    w2 = jnp.max(masked, axis=-1, keepdims=True)
    a2 = jnp.min(jnp.where(masked == w2, iota, E), axis=-1, keepdims=True)

    norm = w1 + w2
    topw_ref[...] = jnp.concatenate([w1 / norm, w2 / norm], axis=-1)
    tope_ref[...] = jnp.concatenate([a1, a2], axis=-1)


def _run_half(xh, W):
    m = xh.shape[0]
    return pl.pallas_call(
            _router_kernel,
            grid=(m // BT,),
            in_specs=[
                pl.BlockSpec((BT, D), lambda i: (i, 0)),
                pl.BlockSpec((E, D), lambda i: (0, 0)),
            ],
            out_specs=[
                pl.BlockSpec((BT, E), lambda i: (i, 0)),
                pl.BlockSpec((BT, 2), lambda i: (i, 0)),
                pl.BlockSpec((BT, 2), lambda i: (i, 0)),
            ],
            out_shape=[
                jax.ShapeDtypeStruct((m, E), jnp.float32),
                jax.ShapeDtypeStruct((m, 2), jnp.float32),
                jax.ShapeDtypeStruct((m, 2), jnp.int32),
            ],
            compiler_params=pltpu.CompilerParams(
                dimension_semantics=("arbitrary",)),
        )(xh, W)


def kernel(x, W):
    xf = x.reshape(-1, x.shape[-1])
    n = xf.shape[0]
    h = n // 2
    wa, ta, ea = _run_half(xf[:h], W)
    wb, tb, eb = _run_half(xf[h:], W)
    return (jnp.concatenate([wa, wb], axis=0),
            jnp.concatenate([ta, tb], axis=0),
            jnp.concatenate([ea, eb], axis=0))


# speedup vs baseline: 2.3146x; 2.3146x over previous
import jax
import jax.numpy as jnp
from jax.experimental import pallas as pl
from jax.experimental.pallas import tpu as pltpu

BT = 1024
E = 16
D = 4096


def _router_kernel(x_ref, w_ref, weights_ref, topw_ref, tope_ref):
    xb = x_ref[...]
    w = w_ref[...]
    logits = jax.lax.dot_general(
        xb, w, (((1,), (1,)), ((), ())),
        preferred_element_type=jnp.float32)

    m1 = jnp.max(logits, axis=-1, keepdims=True)
    s = jnp.exp(logits - m1)
    denom = jnp.sum(s, axis=-1, keepdims=True)
    weights = s / denom
    weights_ref[...] = weights

    iota = jax.lax.broadcasted_iota(jnp.int32, weights.shape, 1)
    w1 = jnp.max(weights, axis=-1, keepdims=True)
    a1 = jnp.min(jnp.where(weights == w1, iota, E), axis=-1, keepdims=True)
    masked = jnp.where(iota == a1, -jnp.inf, weights)
    w2 = jnp.max(masked, axis=-1, keepdims=True)
    a2 = jnp.min(jnp.where(masked == w2, iota, E), axis=-1, keepdims=True)

    norm = w1 + w2
    topw_ref[...] = jnp.concatenate([w1 / norm, w2 / norm], axis=-1)
    tope_ref[...] = jnp.concatenate([a1, a2], axis=-1)


def _run_half(xf, W, blk0, nblk):
    m = nblk * BT
    return pl.pallas_call(
            _router_kernel,
            grid=(nblk,),
            in_specs=[
                pl.BlockSpec((BT, D), lambda i, _b=blk0: (i + _b, 0)),
                pl.BlockSpec((E, D), lambda i: (0, 0)),
            ],
            out_specs=[
                pl.BlockSpec((BT, E), lambda i: (i, 0)),
                pl.BlockSpec((BT, 2), lambda i: (i, 0)),
                pl.BlockSpec((BT, 2), lambda i: (i, 0)),
            ],
            out_shape=[
                jax.ShapeDtypeStruct((m, E), jnp.float32),
                jax.ShapeDtypeStruct((m, 2), jnp.float32),
                jax.ShapeDtypeStruct((m, 2), jnp.int32),
            ],
            compiler_params=pltpu.CompilerParams(
                dimension_semantics=("arbitrary",)),
        )(xf, W)


def kernel(x, W):
    xf = x.reshape(-1, x.shape[-1])
    n = xf.shape[0]
    hblk = n // BT // 2
    wa, ta, ea = _run_half(xf, W, 0, hblk)
    wb, tb, eb = _run_half(xf, W, hblk, hblk)
    return (jnp.concatenate([wa, wb], axis=0),
            jnp.concatenate([ta, tb], axis=0),
            jnp.concatenate([ea, eb], axis=0))
